# baseline (speedup 1.0000x reference)
import jax
import jax.numpy as jnp
from jax import lax
from jax.experimental import pallas as pl
from jax.experimental.pallas import tpu as pltpu
from jax.experimental.pallas import tpu_sc as plsc

_N_NODES = 100000
_N_EDGES = 6400000
_NUM_WORKERS = 32
_PER_TILE = _N_EDGES // _NUM_WORKERS
_CHUNK = 4000
_NCHUNKS = _PER_TILE // _CHUNK
_NPAIRS = _NCHUNKS // 2
_MASK_HI = -65536
_UNROLL = 8


def _pack_table(p, s_half):
    pb = lax.bitcast_convert_type(p.astype(jnp.bfloat16), jnp.uint16)
    sb = lax.bitcast_convert_type(s_half.astype(jnp.bfloat16), jnp.uint16)
    w = (pb.astype(jnp.uint32) << 16) | sb.astype(jnp.uint32)
    return lax.bitcast_convert_type(w, jnp.int32)


def _overlap(w0, w1):
    p0 = plsc.bitcast(w0 & _MASK_HI, jnp.float32)
    p1 = plsc.bitcast(w1 & _MASK_HI, jnp.float32)
    s0 = plsc.bitcast(w0 << 16, jnp.float32)
    s1 = plsc.bitcast(w1 << 16, jnp.float32)
    return jnp.maximum((s0 + s1) - jnp.abs(p0 - p1), 0.0)


def _sc_body(tx_hbm, ty_hbm, ei_hbm, part_hbm,
             spm_x, spm_y,
             i0a, i0b, i1a, i1b,
             gx0a, gx0b, gx1a, gx1b, gy0a, gy0b, gy1a, gy1b,
             part_v,
             sia, sib, sga, sgb):
    cid = lax.axis_index("c")
    sid = lax.axis_index("s")
    wid = cid * 16 + sid
    base_w = wid * _PER_TILE
    ibufs = ((i0a, i1a, sia), (i0b, i1b, sib))
    gbufs = ((gx0a, gx1a, gy0a, gy1a, sga), (gx0b, gx1b, gy0b, gy1b, sgb))

    @pl.when(sid == 0)
    def _():
        pltpu.sync_copy(tx_hbm, spm_x)

    @pl.when(sid == 1)
    def _():
        pltpu.sync_copy(ty_hbm, spm_y)

    plsc.subcore_barrier()

    def start_idx(c, b):
        i0, i1, si = ibufs[b]
        base = base_w + c * _CHUNK
        pltpu.async_copy(ei_hbm.at[pl.ds(base, _CHUNK)], i0, si)
        pltpu.async_copy(ei_hbm.at[pl.ds(_N_EDGES + base, _CHUNK)], i1, si)

    def wait_idx(b):
        i0, i1, si = ibufs[b]
        pltpu.make_async_copy(ei_hbm.at[pl.ds(0, _CHUNK)], i0, si).wait()
        pltpu.make_async_copy(ei_hbm.at[pl.ds(0, _CHUNK)], i1, si).wait()

    def start_gather(b):
        i0, i1, _ = ibufs[b]
        gx0, gx1, gy0, gy1, sg = gbufs[b]
        pltpu.async_copy(spm_x.at[i0], gx0, sg)
        pltpu.async_copy(spm_x.at[i1], gx1, sg)
        pltpu.async_copy(spm_y.at[i0], gy0, sg)
        pltpu.async_copy(spm_y.at[i1], gy1, sg)

    def wait_gather(b):
        i0, i1, _ = ibufs[b]
        gx0, gx1, gy0, gy1, sg = gbufs[b]
        pltpu.make_async_copy(spm_x.at[i0], gx0, sg).wait()
        pltpu.make_async_copy(spm_x.at[i1], gx1, sg).wait()
        pltpu.make_async_copy(spm_y.at[i0], gy0, sg).wait()
        pltpu.make_async_copy(spm_y.at[i1], gy1, sg).wait()

    def compute(b, acc):
        gx0, gx1, gy0, gy1, _ = gbufs[b]

        def jb(off, a):
            ox = _overlap(gx0[pl.ds(off, 16)], gx1[pl.ds(off, 16)])
            oy = _overlap(gy0[pl.ds(off, 16)], gy1[pl.ds(off, 16)])
            return a + ox * oy

        return plsc.parallel_loop(0, _CHUNK, step=16, unroll=_UNROLL,
                                  carry=acc)(jb)

    start_idx(0, 0)
    start_idx(1, 1)
    wait_idx(0)
    start_gather(0)

    def pair(p, acc):
        c0 = p * 2

        wait_idx(1)
        start_gather(1)
        wait_gather(0)

        @pl.when(c0 + 2 < _NCHUNKS)
        def _():
            start_idx(c0 + 2, 0)

        acc = compute(0, acc)

        @pl.when(c0 + 2 < _NCHUNKS)
        def _():
            wait_idx(0)
            start_gather(0)

        wait_gather(1)

        @pl.when(c0 + 3 < _NCHUNKS)
        def _():
            start_idx(c0 + 3, 1)

        acc = compute(1, acc)
        return acc

    acc = lax.fori_loop(0, _NPAIRS, pair, jnp.zeros((16,), jnp.float32))
    part_v[...] = acc
    pltpu.sync_copy(part_v, part_hbm.at[pl.ds(wid * 16, 16)])


@jax.jit
def _sc_call(tx, ty, ei):
    mesh = plsc.VectorSubcoreMesh(core_axis_name="c", subcore_axis_name="s")
    f = pl.kernel(
        _sc_body,
        mesh=mesh,
        out_type=jax.ShapeDtypeStruct((_NUM_WORKERS * 16,), jnp.float32),
        scratch_types=[
            pltpu.VMEM_SHARED((_N_NODES,), jnp.int32),
            pltpu.VMEM_SHARED((_N_NODES,), jnp.int32),
            pltpu.VMEM((_CHUNK,), jnp.int32),
            pltpu.VMEM((_CHUNK,), jnp.int32),
            pltpu.VMEM((_CHUNK,), jnp.int32),
            pltpu.VMEM((_CHUNK,), jnp.int32),
            pltpu.VMEM((_CHUNK,), jnp.int32),
            pltpu.VMEM((_CHUNK,), jnp.int32),
            pltpu.VMEM((_CHUNK,), jnp.int32),
            pltpu.VMEM((_CHUNK,), jnp.int32),
            pltpu.VMEM((_CHUNK,), jnp.int32),
            pltpu.VMEM((_CHUNK,), jnp.int32),
            pltpu.VMEM((_CHUNK,), jnp.int32),
            pltpu.VMEM((_CHUNK,), jnp.int32),
            pltpu.VMEM((16,), jnp.float32),
            pltpu.SemaphoreType.DMA,
            pltpu.SemaphoreType.DMA,
            pltpu.SemaphoreType.DMA,
            pltpu.SemaphoreType.DMA,
        ],
        compiler_params=pltpu.CompilerParams(needs_layout_passes=False),
    )
    return f(tx, ty, ei)


def kernel(node_pos, node_sizes, edge_index):
    tx = _pack_table(node_pos[:, 0], node_sizes[:, 0] * 0.5)
    ty = _pack_table(node_pos[:, 1], node_sizes[:, 1] * 0.5)
    part = _sc_call(tx, ty, edge_index.reshape(-1))
    return jnp.sum(part) / jnp.float32(_N_EDGES)


# speedup vs baseline: 1.7226x; 1.7226x over previous
import functools

import jax
import jax.numpy as jnp
from jax import lax
from jax.experimental import pallas as pl
from jax.experimental.pallas import tpu as pltpu
from jax.experimental.pallas import tpu_sc as plsc

_N_NODES = 100000
_N_EDGES = 6400000
_NUM_WORKERS = 32
_PER_TILE = _N_EDGES // _NUM_WORKERS
_CHUNK = 4000
_NCHUNKS = _PER_TILE // _CHUNK
_NPAIRS = _NCHUNKS // 2
_VPC = _CHUNK // 32
_MASK_HI = -65536
_UNROLL = 8


def _pack_table(p, s_half):
    pb = lax.bitcast_convert_type(p.astype(jnp.bfloat16), jnp.uint16)
    sb = lax.bitcast_convert_type(s_half.astype(jnp.bfloat16), jnp.uint16)
    w = (pb.astype(jnp.uint32) << 16) | sb.astype(jnp.uint32)
    return lax.bitcast_convert_type(w, jnp.int32)


def _overlap_vec(tab_ref, idx0_ref, idx1_ref, off):
    i0 = idx0_ref[pl.ds(off, 16)]
    i1 = idx1_ref[pl.ds(off, 16)]
    w0 = plsc.load_gather(tab_ref, [i0])
    w1 = plsc.load_gather(tab_ref, [i1])
    p0 = plsc.bitcast(w0 & _MASK_HI, jnp.float32)
    p1 = plsc.bitcast(w1 & _MASK_HI, jnp.float32)
    s0 = plsc.bitcast(w0 << 16, jnp.float32)
    s1 = plsc.bitcast(w1 << 16, jnp.float32)
    return jnp.maximum((s0 + s1) - jnp.abs(p0 - p1), 0.0)


def _sc_body(tx_hbm, ty_hbm, ei_hbm, part_hbm, spill_hbm,
             tab_v, i0a, i0b, i1a, i1b, ova, ovb, part_v,
             s0a, s0b, s1a, s1b, sva, svb):
    wid = lax.axis_index("c") * 16 + lax.axis_index("s")
    base_w = wid * _PER_TILE
    ibufs = ((i0a, i1a, s0a, s1a), (i0b, i1b, s0b, s1b))
    obufs = ((ova, sva), (ovb, svb))

    def start_in(c, b):
        i0, i1, se0, se1 = ibufs[b]
        base = base_w + c * _CHUNK
        pltpu.async_copy(ei_hbm.at[pl.ds(base, _CHUNK)], i0, se0)
        pltpu.async_copy(ei_hbm.at[pl.ds(_N_EDGES + base, _CHUNK)], i1, se1)

    def wait_in(b):
        i0, i1, se0, se1 = ibufs[b]
        pltpu.make_async_copy(ei_hbm.at[pl.ds(0, _CHUNK)], i0, se0).wait()
        pltpu.make_async_copy(ei_hbm.at[pl.ds(0, _CHUNK)], i1, se1).wait()

    def start_spill_out(c, b):
        ov, sv = obufs[b]
        base = wid * (_PER_TILE // 2) + c * (_CHUNK // 2)
        pltpu.async_copy(ov, spill_hbm.at[pl.ds(base, _CHUNK // 2)], sv)

    def start_spill_in(c, b):
        ov, sv = obufs[b]
        base = wid * (_PER_TILE // 2) + c * (_CHUNK // 2)
        pltpu.async_copy(spill_hbm.at[pl.ds(base, _CHUNK // 2)], ov, sv)

    def wait_spill(b):
        ov, sv = obufs[b]
        pltpu.make_async_copy(
            spill_hbm.at[pl.ds(0, _CHUNK // 2)], ov, sv).wait()

    pltpu.sync_copy(tx_hbm, tab_v)
    start_in(0, 0)

    def compute1(b):
        i0, i1, _, _ = ibufs[b]
        ov, _ = obufs[b]

        @plsc.parallel_loop(0, _CHUNK, step=32, unroll=_UNROLL)
        def _(off):
            ox0 = _overlap_vec(tab_v, i0, i1, off)
            ox1 = _overlap_vec(tab_v, i0, i1, off + 16)
            packed = plsc.pack(ox0, ox1, format=plsc.PackFormat.INTERLEAVED)
            off2 = pl.multiple_of(off // 2, 16)
            ov[pl.ds(off2, 16)] = plsc.bitcast(packed, jnp.int32)

    def pair1(p, carry):
        c0 = p * 2
        start_in(c0 + 1, 1)
        wait_in(0)

        @pl.when(p > 0)
        def _():
            wait_spill(0)

        compute1(0)
        start_spill_out(c0, 0)

        @pl.when(p < _NPAIRS - 1)
        def _():
            start_in(c0 + 2, 0)

        wait_in(1)

        @pl.when(p > 0)
        def _():
            wait_spill(1)

        compute1(1)
        start_spill_out(c0 + 1, 1)
        return carry

    lax.fori_loop(0, _NPAIRS, pair1, 0)
    wait_spill(0)
    wait_spill(1)

    pltpu.sync_copy(ty_hbm, tab_v)
    start_in(0, 0)
    start_spill_in(0, 0)

    def compute2(b, acc):
        i0, i1, _, _ = ibufs[b]
        ov, _ = obufs[b]

        def jb(off, accs):
            aa, ab = accs
            oy0 = _overlap_vec(tab_v, i0, i1, off)
            oy1 = _overlap_vec(tab_v, i0, i1, off + 16)
            off2 = pl.multiple_of(off // 2, 16)
            packed = plsc.bitcast(ov[pl.ds(off2, 16)], jnp.bfloat16)
            ox0, ox1 = plsc.unpack(packed, format=plsc.PackFormat.INTERLEAVED)
            aa = aa + oy0 * ox0
            ab = ab + oy1 * ox1
            return (aa, ab)

        return plsc.parallel_loop(0, _CHUNK, step=32, unroll=_UNROLL,
                                  carry=acc)(jb)

    def pair2(p, acc):
        c0 = p * 2
        start_in(c0 + 1, 1)
        start_spill_in(c0 + 1, 1)
        wait_in(0)
        wait_spill(0)
        acc = compute2(0, acc)

        @pl.when(p < _NPAIRS - 1)
        def _():
            start_in(c0 + 2, 0)
            start_spill_in(c0 + 2, 0)

        wait_in(1)
        wait_spill(1)
        acc = compute2(1, acc)
        return acc

    zero = jnp.zeros((16,), jnp.float32)
    acc_a, acc_b = lax.fori_loop(0, _NPAIRS, pair2, (zero, zero))
    part_v[...] = acc_a + acc_b
    pltpu.sync_copy(part_v, part_hbm.at[pl.ds(wid * 16, 16)])


@jax.jit
def _sc_call(tx, ty, ei):
    mesh = plsc.VectorSubcoreMesh(core_axis_name="c", subcore_axis_name="s")
    f = pl.kernel(
        _sc_body,
        mesh=mesh,
        out_type=(
            jax.ShapeDtypeStruct((_NUM_WORKERS * 16,), jnp.float32),
            jax.ShapeDtypeStruct((_N_EDGES // 2,), jnp.int32),
        ),
        scratch_types=[
            pltpu.VMEM((_N_NODES,), jnp.int32),
            pltpu.VMEM((_CHUNK,), jnp.int32),
            pltpu.VMEM((_CHUNK,), jnp.int32),
            pltpu.VMEM((_CHUNK,), jnp.int32),
            pltpu.VMEM((_CHUNK,), jnp.int32),
            pltpu.VMEM((_CHUNK // 2,), jnp.int32),
            pltpu.VMEM((_CHUNK // 2,), jnp.int32),
            pltpu.VMEM((16,), jnp.float32),
            pltpu.SemaphoreType.DMA,
            pltpu.SemaphoreType.DMA,
            pltpu.SemaphoreType.DMA,
            pltpu.SemaphoreType.DMA,
            pltpu.SemaphoreType.DMA,
            pltpu.SemaphoreType.DMA,
        ],
        compiler_params=pltpu.CompilerParams(needs_layout_passes=False),
    )
    return f(tx, ty, ei)


def kernel(node_pos, node_sizes, edge_index):
    tx = _pack_table(node_pos[:, 0], node_sizes[:, 0] * 0.5)
    ty = _pack_table(node_pos[:, 1], node_sizes[:, 1] * 0.5)
    part, _ = _sc_call(tx, ty, edge_index.reshape(-1))
    return jnp.sum(part) / jnp.float32(_N_EDGES)


# speedup vs baseline: 1.7589x; 1.0211x over previous
import functools

import jax
import jax.numpy as jnp
from jax import lax
from jax.experimental import pallas as pl
from jax.experimental.pallas import tpu as pltpu
from jax.experimental.pallas import tpu_sc as plsc

_N_NODES = 100000
_N_EDGES = 6400000
_NUM_WORKERS = 32
_PER_TILE = _N_EDGES // _NUM_WORKERS
_CHUNK = 4000
_NCHUNKS = _PER_TILE // _CHUNK
_NPAIRS = _NCHUNKS // 2
_VPC = _CHUNK // 32
_MASK_HI = -65536
_UNROLL = 4


def _pack_table(p, s_half):
    pb = lax.bitcast_convert_type(p.astype(jnp.bfloat16), jnp.uint16)
    sb = lax.bitcast_convert_type(s_half.astype(jnp.bfloat16), jnp.uint16)
    w = (pb.astype(jnp.uint32) << 16) | sb.astype(jnp.uint32)
    return lax.bitcast_convert_type(w, jnp.int32)


def _overlap_vec(tab_ref, idx0_ref, idx1_ref, off):
    i0 = idx0_ref[pl.ds(off, 16)]
    i1 = idx1_ref[pl.ds(off, 16)]
    w0 = plsc.load_gather(tab_ref, [i0])
    w1 = plsc.load_gather(tab_ref, [i1])
    w0b = plsc.bitcast(w0, jnp.bfloat16)
    w1b = plsc.bitcast(w1, jnp.bfloat16)
    di = plsc.bitcast(jnp.abs(w0b - w1b), jnp.int32)
    si = plsc.bitcast(w0b + w1b, jnp.int32)
    dp = plsc.bitcast(di & _MASK_HI, jnp.float32)
    ss = plsc.bitcast(si << 16, jnp.float32)
    return jnp.maximum(ss - dp, 0.0)


def _sc_body(tx_hbm, ty_hbm, ei_hbm, part_hbm, spill_hbm,
             tab_v, i0a, i0b, i1a, i1b, ova, ovb, part_v,
             s0a, s0b, s1a, s1b, sva, svb):
    wid = lax.axis_index("c") * 16 + lax.axis_index("s")
    base_w = wid * _PER_TILE
    ibufs = ((i0a, i1a, s0a, s1a), (i0b, i1b, s0b, s1b))
    obufs = ((ova, sva), (ovb, svb))

    def start_in(c, b):
        i0, i1, se0, se1 = ibufs[b]
        base = base_w + c * _CHUNK
        pltpu.async_copy(ei_hbm.at[pl.ds(base, _CHUNK)], i0, se0)
        pltpu.async_copy(ei_hbm.at[pl.ds(_N_EDGES + base, _CHUNK)], i1, se1)

    def wait_in(b):
        i0, i1, se0, se1 = ibufs[b]
        pltpu.make_async_copy(ei_hbm.at[pl.ds(0, _CHUNK)], i0, se0).wait()
        pltpu.make_async_copy(ei_hbm.at[pl.ds(0, _CHUNK)], i1, se1).wait()

    def start_spill_out(c, b):
        ov, sv = obufs[b]
        base = wid * (_PER_TILE // 2) + c * (_CHUNK // 2)
        pltpu.async_copy(ov, spill_hbm.at[pl.ds(base, _CHUNK // 2)], sv)

    def start_spill_in(c, b):
        ov, sv = obufs[b]
        base = wid * (_PER_TILE // 2) + c * (_CHUNK // 2)
        pltpu.async_copy(spill_hbm.at[pl.ds(base, _CHUNK // 2)], ov, sv)

    def wait_spill(b):
        ov, sv = obufs[b]
        pltpu.make_async_copy(
            spill_hbm.at[pl.ds(0, _CHUNK // 2)], ov, sv).wait()

    pltpu.sync_copy(tx_hbm, tab_v)
    start_in(0, 0)

    def compute1(b):
        i0, i1, _, _ = ibufs[b]
        ov, _ = obufs[b]

        @plsc.parallel_loop(0, _CHUNK, step=32, unroll=_UNROLL)
        def _(off):
            ox0 = _overlap_vec(tab_v, i0, i1, off)
            ox1 = _overlap_vec(tab_v, i0, i1, off + 16)
            packed = plsc.pack(ox0, ox1, format=plsc.PackFormat.INTERLEAVED)
            off2 = pl.multiple_of(off // 2, 16)
            ov[pl.ds(off2, 16)] = plsc.bitcast(packed, jnp.int32)

    def pair1(p, carry):
        c0 = p * 2
        start_in(c0 + 1, 1)
        wait_in(0)

        @pl.when(p > 0)
        def _():
            wait_spill(0)

        compute1(0)
        start_spill_out(c0, 0)

        @pl.when(p < _NPAIRS - 1)
        def _():
            start_in(c0 + 2, 0)

        wait_in(1)

        @pl.when(p > 0)
        def _():
            wait_spill(1)

        compute1(1)
        start_spill_out(c0 + 1, 1)
        return carry

    lax.fori_loop(0, _NPAIRS, pair1, 0)
    wait_spill(0)
    wait_spill(1)

    pltpu.sync_copy(ty_hbm, tab_v)
    start_in(0, 0)
    start_spill_in(0, 0)

    def compute2(b, acc):
        i0, i1, _, _ = ibufs[b]
        ov, _ = obufs[b]

        def jb(off, accs):
            aa, ab = accs
            oy0 = _overlap_vec(tab_v, i0, i1, off)
            oy1 = _overlap_vec(tab_v, i0, i1, off + 16)
            off2 = pl.multiple_of(off // 2, 16)
            packed = plsc.bitcast(ov[pl.ds(off2, 16)], jnp.bfloat16)
            ox0, ox1 = plsc.unpack(packed, format=plsc.PackFormat.INTERLEAVED)
            aa = aa + oy0 * ox0
            ab = ab + oy1 * ox1
            return (aa, ab)

        return plsc.parallel_loop(0, _CHUNK, step=32, unroll=_UNROLL,
                                  carry=acc)(jb)

    def pair2(p, acc):
        c0 = p * 2
        start_in(c0 + 1, 1)
        start_spill_in(c0 + 1, 1)
        wait_in(0)
        wait_spill(0)
        acc = compute2(0, acc)

        @pl.when(p < _NPAIRS - 1)
        def _():
            start_in(c0 + 2, 0)
            start_spill_in(c0 + 2, 0)

        wait_in(1)
        wait_spill(1)
        acc = compute2(1, acc)
        return acc

    zero = jnp.zeros((16,), jnp.float32)
    acc_a, acc_b = lax.fori_loop(0, _NPAIRS, pair2, (zero, zero))
    part_v[...] = acc_a + acc_b
    pltpu.sync_copy(part_v, part_hbm.at[pl.ds(wid * 16, 16)])


@jax.jit
def _sc_call(tx, ty, ei):
    mesh = plsc.VectorSubcoreMesh(core_axis_name="c", subcore_axis_name="s")
    f = pl.kernel(
        _sc_body,
        mesh=mesh,
        out_type=(
            jax.ShapeDtypeStruct((_NUM_WORKERS * 16,), jnp.float32),
            jax.ShapeDtypeStruct((_N_EDGES // 2,), jnp.int32),
        ),
        scratch_types=[
            pltpu.VMEM((_N_NODES,), jnp.int32),
            pltpu.VMEM((_CHUNK,), jnp.int32),
            pltpu.VMEM((_CHUNK,), jnp.int32),
            pltpu.VMEM((_CHUNK,), jnp.int32),
            pltpu.VMEM((_CHUNK,), jnp.int32),
            pltpu.VMEM((_CHUNK // 2,), jnp.int32),
            pltpu.VMEM((_CHUNK // 2,), jnp.int32),
            pltpu.VMEM((16,), jnp.float32),
            pltpu.SemaphoreType.DMA,
            pltpu.SemaphoreType.DMA,
            pltpu.SemaphoreType.DMA,
            pltpu.SemaphoreType.DMA,
            pltpu.SemaphoreType.DMA,
            pltpu.SemaphoreType.DMA,
        ],
        compiler_params=pltpu.CompilerParams(needs_layout_passes=False),
    )
    return f(tx, ty, ei)


def kernel(node_pos, node_sizes, edge_index):
    tx = _pack_table(node_pos[:, 0], node_sizes[:, 0] * 0.5)
    ty = _pack_table(node_pos[:, 1], node_sizes[:, 1] * 0.5)
    part, _ = _sc_call(tx, ty, edge_index.reshape(-1))
    return jnp.sum(part) / jnp.float32(_N_EDGES)


# speedup vs baseline: 1.7614x; 1.0014x over previous
import jax
import jax.numpy as jnp
from jax import lax
from jax.experimental import pallas as pl
from jax.experimental.pallas import tpu as pltpu
from jax.experimental.pallas import tpu_sc as plsc

_N_NODES = 100000
_N_EDGES = 6400000
_NUM_WORKERS = 32
_PER_TILE = _N_EDGES // _NUM_WORKERS
_CHUNK = 4000
_NCHUNKS = _PER_TILE // _CHUNK
_NPAIRS = _NCHUNKS // 2
_VPC = _CHUNK // 32
_MASK_HI = -65536
_UNROLL = 4


def _pack_table(p, s_half):
    pb = lax.bitcast_convert_type(p.astype(jnp.bfloat16), jnp.uint16)
    sb = lax.bitcast_convert_type(s_half.astype(jnp.bfloat16), jnp.uint16)
    w = (pb.astype(jnp.uint32) << 16) | sb.astype(jnp.uint32)
    return lax.bitcast_convert_type(w, jnp.int32)


def _overlap_vec(tab_ref, idx0_ref, idx1_ref, off):
    i0 = idx0_ref[pl.ds(off, 16)]
    i1 = idx1_ref[pl.ds(off, 16)]
    w0 = plsc.load_gather(tab_ref, [i0])
    w1 = plsc.load_gather(tab_ref, [i1])
    w0b = plsc.bitcast(w0, jnp.bfloat16)
    w1b = plsc.bitcast(w1, jnp.bfloat16)
    di = plsc.bitcast(jnp.abs(w0b - w1b), jnp.int32)
    si = plsc.bitcast(w0b + w1b, jnp.int32)
    dp = plsc.bitcast(di & _MASK_HI, jnp.float32)
    ss = plsc.bitcast(si << 16, jnp.float32)
    return jnp.maximum(ss - dp, 0.0)


def _sc_body(tx_hbm, ty_hbm, ei_hbm, part_hbm, spill_hbm,
             tab_v, i0a, i0b, i1a, i1b, ova, ovb, part_v,
             s0a, s0b, s1a, s1b, sva, svb):
    wid = lax.axis_index("c") * 16 + lax.axis_index("s")
    base_w = wid * _PER_TILE
    ibufs = ((i0a, i1a, s0a, s1a), (i0b, i1b, s0b, s1b))
    obufs = ((ova, sva), (ovb, svb))

    def start_in(c, b):
        i0, i1, se0, se1 = ibufs[b]
        base = base_w + c * _CHUNK
        pltpu.async_copy(ei_hbm.at[pl.ds(base, _CHUNK)], i0, se0)
        pltpu.async_copy(ei_hbm.at[pl.ds(_N_EDGES + base, _CHUNK)], i1, se1)

    def wait_in(b):
        i0, i1, se0, se1 = ibufs[b]
        pltpu.make_async_copy(ei_hbm.at[pl.ds(0, _CHUNK)], i0, se0).wait()
        pltpu.make_async_copy(ei_hbm.at[pl.ds(0, _CHUNK)], i1, se1).wait()

    def start_spill_out(c, b):
        ov, sv = obufs[b]
        base = wid * (_PER_TILE // 2) + c * (_CHUNK // 2)
        pltpu.async_copy(ov, spill_hbm.at[pl.ds(base, _CHUNK // 2)], sv)

    def start_spill_in(c, b):
        ov, sv = obufs[b]
        base = wid * (_PER_TILE // 2) + c * (_CHUNK // 2)
        pltpu.async_copy(spill_hbm.at[pl.ds(base, _CHUNK // 2)], ov, sv)

    def wait_spill(b):
        ov, sv = obufs[b]
        pltpu.make_async_copy(
            spill_hbm.at[pl.ds(0, _CHUNK // 2)], ov, sv).wait()

    pltpu.sync_copy(tx_hbm, tab_v)
    start_in(0, 0)

    def compute1(b):
        i0, i1, _, _ = ibufs[b]
        ov, _ = obufs[b]

        @plsc.parallel_loop(0, _CHUNK, step=32, unroll=_UNROLL)
        def _(off):
            ox0 = _overlap_vec(tab_v, i0, i1, off)
            ox1 = _overlap_vec(tab_v, i0, i1, off + 16)
            packed = plsc.pack(ox0, ox1, format=plsc.PackFormat.INTERLEAVED)
            off2 = pl.multiple_of(off // 2, 16)
            ov[pl.ds(off2, 16)] = plsc.bitcast(packed, jnp.int32)

    def pair1(p, carry):
        c0 = p * 2
        start_in(c0 + 1, 1)
        wait_in(0)

        @pl.when(p > 0)
        def _():
            wait_spill(0)

        compute1(0)
        start_spill_out(c0, 0)

        @pl.when(p < _NPAIRS - 1)
        def _():
            start_in(c0 + 2, 0)

        wait_in(1)

        @pl.when(p > 0)
        def _():
            wait_spill(1)

        compute1(1)
        start_spill_out(c0 + 1, 1)
        return carry

    lax.fori_loop(0, _NPAIRS, pair1, 0)
    wait_spill(0)
    wait_spill(1)

    pltpu.sync_copy(ty_hbm, tab_v)
    start_in(0, 0)
    start_spill_in(0, 0)

    def compute2(b, acc):
        i0, i1, _, _ = ibufs[b]
        ov, _ = obufs[b]

        def jb(off, accs):
            aa, ab = accs
            oy0 = _overlap_vec(tab_v, i0, i1, off)
            oy1 = _overlap_vec(tab_v, i0, i1, off + 16)
            off2 = pl.multiple_of(off // 2, 16)
            packed = plsc.bitcast(ov[pl.ds(off2, 16)], jnp.bfloat16)
            ox0, ox1 = plsc.unpack(packed, format=plsc.PackFormat.INTERLEAVED)
            aa = aa + oy0 * ox0
            ab = ab + oy1 * ox1
            return (aa, ab)

        return plsc.parallel_loop(0, _CHUNK, step=32, unroll=_UNROLL,
                                  carry=acc)(jb)

    def pair2(p, acc):
        c0 = p * 2
        start_in(c0 + 1, 1)
        start_spill_in(c0 + 1, 1)
        wait_in(0)
        wait_spill(0)
        acc = compute2(0, acc)

        @pl.when(p < _NPAIRS - 1)
        def _():
            start_in(c0 + 2, 0)
            start_spill_in(c0 + 2, 0)

        wait_in(1)
        wait_spill(1)
        acc = compute2(1, acc)
        return acc

    zero = jnp.zeros((16,), jnp.float32)
    acc_a, acc_b = lax.fori_loop(0, _NPAIRS, pair2, (zero, zero))
    part_v[...] = acc_a + acc_b
    pltpu.sync_copy(part_v, part_hbm.at[pl.ds(wid * 16, 16)])


@jax.jit
def _sc_call(tx, ty, ei):
    mesh = plsc.VectorSubcoreMesh(core_axis_name="c", subcore_axis_name="s")
    f = pl.kernel(
        _sc_body,
        mesh=mesh,
        out_type=(
            jax.ShapeDtypeStruct((_NUM_WORKERS * 16,), jnp.float32),
            jax.ShapeDtypeStruct((_N_EDGES // 2,), jnp.int32),
        ),
        scratch_types=[
            pltpu.VMEM((_N_NODES,), jnp.int32),
            pltpu.VMEM((_CHUNK,), jnp.int32),
            pltpu.VMEM((_CHUNK,), jnp.int32),
            pltpu.VMEM((_CHUNK,), jnp.int32),
            pltpu.VMEM((_CHUNK,), jnp.int32),
            pltpu.VMEM((_CHUNK // 2,), jnp.int32),
            pltpu.VMEM((_CHUNK // 2,), jnp.int32),
            pltpu.VMEM((16,), jnp.float32),
            pltpu.SemaphoreType.DMA,
            pltpu.SemaphoreType.DMA,
            pltpu.SemaphoreType.DMA,
            pltpu.SemaphoreType.DMA,
            pltpu.SemaphoreType.DMA,
            pltpu.SemaphoreType.DMA,
        ],
        compiler_params=pltpu.CompilerParams(needs_layout_passes=False),
    )
    return f(tx, ty, ei)


def kernel(node_pos, node_sizes, edge_index):
    tx = _pack_table(node_pos[:, 0], node_sizes[:, 0] * 0.5)
    ty = _pack_table(node_pos[:, 1], node_sizes[:, 1] * 0.5)
    part, _ = _sc_call(tx, ty, edge_index.reshape(-1))
    return jnp.sum(part) / jnp.float32(_N_EDGES)
